# baseline (speedup 1.0000x reference)
import functools

import jax
import jax.numpy as jnp
from jax import lax
from jax.experimental import pallas as pl
from jax.experimental.pallas import tpu as pltpu
from jax.experimental.pallas import tpu_sc as plsc

NTOKEN = 10000
NINP = 256
NHID = 512
N = 256
R = 4
W = 64
BATCH = 64
SEQ = 20
XI = R * W + R + W + 1 + W + W + R + 1 + 1 + 3 * R

BB = 8
EPS = 1e-6


def _softplus(x):
    return jnp.maximum(x, 0.0) + jnp.log1p(jnp.exp(-jnp.abs(x)))


def _prod_last(x):
    w = x.shape[-1]
    while w > 1:
        h = w // 2
        x = x[..., :h] * x[..., h:w]
        w = h
    return x[..., 0]


def _softmax(x):
    m = jnp.max(x, axis=-1, keepdims=True)
    e = jnp.exp(x - m)
    return e / jnp.sum(e, axis=-1, keepdims=True)


def _sc_gather(table, idx):
    info = plsc.get_sparse_core_info()
    nw = info.num_cores * info.num_subcores
    b = idx.shape[0]
    b_per_w = b // nw
    mesh = plsc.VectorSubcoreMesh(core_axis_name="c", subcore_axis_name="s")

    @functools.partial(
        pl.kernel, mesh=mesh,
        out_type=jax.ShapeDtypeStruct((b, NINP), jnp.float32),
        scratch_types=[
            pltpu.VMEM((b_per_w,), jnp.int32),
            pltpu.VMEM((b_per_w, NINP), jnp.float32),
            pltpu.SemaphoreType.DMA,
        ],
    )
    def k(table_hbm, idx_hbm, out_hbm, idx_v, rows_v, sem):
        wid = lax.axis_index("s") * info.num_cores + lax.axis_index("c")
        base = wid * b_per_w
        pltpu.sync_copy(idx_hbm.at[pl.ds(base, b_per_w)], idx_v)
        pltpu.async_copy(table_hbm.at[idx_v], rows_v, sem).wait()
        pltpu.sync_copy(rows_v, out_hbm.at[pl.ds(base, b_per_w)])

    return k(table, idx)


def _scan_kernel(emb_ref, h0_ref, W_ih_ref, W_hh_ref, b_h_ref, W_xi_ref,
                 b_xi_ref, W_out_ref, b_out_ref, outs_ref,
                 mem_ref, link_ref, prec_ref, usage_ref, ww_ref, rw_ref,
                 rv_ref, h_ref):
    f32 = jnp.float32
    h_ref[...] = h0_ref[...]
    mem_ref[...] = jnp.full((BB, N, W), 1e-6, f32)
    link_ref[...] = jnp.zeros((BB, N, N), f32)
    prec_ref[...] = jnp.zeros((BB, N), f32)
    usage_ref[...] = jnp.zeros((BB, N), f32)
    ww_ref[...] = jnp.zeros((BB, N), f32)
    rw_ref[...] = jnp.zeros((BB, R, N), f32)
    rv_ref[...] = jnp.zeros((BB, R * W), f32)

    ii = lax.broadcasted_iota(jnp.int32, (N, N), 0)
    jj = lax.broadcasted_iota(jnp.int32, (N, N), 1)
    offdiag = jnp.where(ii == jj, 0.0, 1.0)

    def step(t, _):
        x_t = emb_ref[t]
        rw_old = rw_ref[...]
        usage = usage_ref[...]
        ww_prev = ww_ref[...]
        mem = mem_ref[...]
        link = link_ref[...]
        prec = prec_ref[...]

        cin = jnp.concatenate([x_t, rv_ref[...]], axis=1)
        h = jnp.tanh(
            jnp.dot(cin, W_ih_ref[...], preferred_element_type=f32)
            + jnp.dot(h_ref[...], W_hh_ref[...], preferred_element_type=f32)
            + b_h_ref[...])
        xi = jnp.dot(h, W_xi_ref[...], preferred_element_type=f32) + b_xi_ref[...]

        o = 0
        read_keys = xi[:, o:o + R * W]; o += R * W
        read_str = 1.0 + _softplus(xi[:, o:o + R]); o += R
        write_key = xi[:, o:o + W]; o += W
        write_str = 1.0 + _softplus(xi[:, o:o + 1]); o += 1
        erase = jax.nn.sigmoid(xi[:, o:o + W]); o += W
        write_vec = xi[:, o:o + W]; o += W
        free = jax.nn.sigmoid(xi[:, o:o + R]); o += R
        alloc_gate = jax.nn.sigmoid(xi[:, o:o + 1]); o += 1
        write_gate = jax.nn.sigmoid(xi[:, o:o + 1]); o += 1
        modes_raw = xi[:, o:o + 3 * R]

        ret = jnp.ones((BB, N), f32)
        for rr in range(R):
            ret = ret * (1.0 - free[:, rr:rr + 1] * rw_old[:, rr, :])
        usage = (usage + ww_prev - usage * ww_prev) * ret

        ui = usage[:, :, None]
        uj = usage[:, None, :]
        before = jnp.logical_or(uj < ui, jnp.logical_and(uj == ui, jj < ii))
        vals = jnp.where(before, jnp.broadcast_to(uj, (BB, N, N)), 1.0)
        cp = _prod_last(vals)
        alloc = (1.0 - usage) * cp

        mem_norm = jnp.sqrt(jnp.sum(mem * mem, axis=2))
        wdot = jnp.sum(mem * write_key[:, None, :], axis=2)
        wk_norm = jnp.sqrt(jnp.sum(write_key * write_key, axis=1, keepdims=True))
        cw = _softmax(write_str * (wdot / (wk_norm * mem_norm + EPS)))
        ww = write_gate * (alloc_gate * alloc + (1.0 - alloc_gate) * cw)

        mem = (mem * (1.0 - ww[:, :, None] * erase[:, None, :])
               + ww[:, :, None] * write_vec[:, None, :])

        link = ((1.0 - ww[:, :, None] - ww[:, None, :]) * link
                + ww[:, :, None] * prec[:, None, :])
        link = link * offdiag[None]
        prec = (1.0 - jnp.sum(ww, axis=1, keepdims=True)) * prec + ww

        mem_norm2 = jnp.sqrt(jnp.sum(mem * mem, axis=2))
        new_rw = []
        new_rv = []
        for rr in range(R):
            rwr = rw_old[:, rr, :]
            fwd = jnp.sum(link * rwr[:, None, :], axis=2)
            bwd = jnp.sum(link * rwr[:, :, None], axis=1)
            key = read_keys[:, W * rr:W * rr + W]
            kdot = jnp.sum(mem * key[:, None, :], axis=2)
            kn = jnp.sqrt(jnp.sum(key * key, axis=1, keepdims=True))
            cr = _softmax(read_str[:, rr:rr + 1] * (kdot / (kn * mem_norm2 + EPS)))
            m = _softmax(modes_raw[:, 3 * rr:3 * rr + 3])
            nrw = (m[:, 0:1] * bwd + m[:, 1:2] * cr + m[:, 2:3] * fwd)
            new_rw.append(nrw[:, None, :])
            new_rv.append(jnp.sum(mem * nrw[:, :, None], axis=1))

        rw_new = jnp.concatenate(new_rw, axis=1)
        rv_flat = jnp.concatenate(new_rv, axis=1)

        out = (jnp.dot(jnp.concatenate([h, rv_flat], axis=1), W_out_ref[...],
                       preferred_element_type=f32) + b_out_ref[...])
        outs_ref[t] = out

        mem_ref[...] = mem
        link_ref[...] = link
        prec_ref[...] = prec
        usage_ref[...] = usage
        ww_ref[...] = ww
        rw_ref[...] = rw_new
        rv_ref[...] = rv_flat
        h_ref[...] = h
        return 0

    lax.fori_loop(0, SEQ, step, 0)


def _run_scan(emb, h0, W_ih, W_hh, b_h, W_xi, b_xi, W_out, b_out):
    f32 = jnp.float32
    nb = BATCH // BB
    full = lambda i: (0, 0)
    return pl.pallas_call(
        _scan_kernel,
        grid=(nb,),
        in_specs=[
            pl.BlockSpec((SEQ, BB, NINP), lambda i: (0, i, 0)),
            pl.BlockSpec((BB, NHID), lambda i: (i, 0)),
            pl.BlockSpec((NINP + R * W, NHID), full),
            pl.BlockSpec((NHID, NHID), full),
            pl.BlockSpec((1, NHID), full),
            pl.BlockSpec((NHID, XI), full),
            pl.BlockSpec((1, XI), full),
            pl.BlockSpec((NHID + R * W, NINP), full),
            pl.BlockSpec((1, NINP), full),
        ],
        out_specs=pl.BlockSpec((SEQ, BB, NINP), lambda i: (0, i, 0)),
        out_shape=jax.ShapeDtypeStruct((SEQ, BATCH, NINP), f32),
        scratch_shapes=[
            pltpu.VMEM((BB, N, W), f32),
            pltpu.VMEM((BB, N, N), f32),
            pltpu.VMEM((BB, N), f32),
            pltpu.VMEM((BB, N), f32),
            pltpu.VMEM((BB, N), f32),
            pltpu.VMEM((BB, R, N), f32),
            pltpu.VMEM((BB, R * W), f32),
            pltpu.VMEM((BB, NHID), f32),
        ],
    )(emb, h0, W_ih, W_hh, b_h, W_xi, b_xi, W_out, b_out)


def _decode_kernel(x_ref, w_ref, b_ref, o_ref):
    o_ref[...] = (jnp.dot(x_ref[...], w_ref[...],
                          preferred_element_type=jnp.float32) + b_ref[...])


def _run_decode(x2d, W_dec, b_dec):
    cb = 1280
    nb = pl.cdiv(NTOKEN, cb)
    rows = x2d.shape[0]
    return pl.pallas_call(
        _decode_kernel,
        grid=(nb,),
        in_specs=[
            pl.BlockSpec((rows, NINP), lambda j: (0, 0)),
            pl.BlockSpec((NINP, cb), lambda j: (0, j)),
            pl.BlockSpec((1, cb), lambda j: (0, j)),
        ],
        out_specs=pl.BlockSpec((rows, cb), lambda j: (0, j)),
        out_shape=jax.ShapeDtypeStruct((rows, NTOKEN), jnp.float32),
    )(x2d, W_dec, b_dec)


def kernel(input, hidden, encoder_w, W_ih, W_hh, b_h, W_xi, b_xi, W_out,
           b_out, W_dec, b_dec):
    idx_tm = jnp.transpose(input).reshape(SEQ * BATCH)
    emb2d = _sc_gather(encoder_w, idx_tm)
    emb = emb2d.reshape(SEQ, BATCH, NINP)

    outs = _run_scan(
        emb, hidden[0], W_ih, W_hh, b_h.reshape(1, NHID), W_xi,
        b_xi.reshape(1, XI), W_out, b_out.reshape(1, NINP))

    x2d = jnp.transpose(outs, (1, 0, 2)).reshape(BATCH * SEQ, NINP)
    decoded = _run_decode(x2d, W_dec, b_dec.reshape(1, NTOKEN))
    return decoded.reshape(BATCH, SEQ, NTOKEN)


# speedup vs baseline: 1.1480x; 1.1480x over previous
import functools

import jax
import jax.numpy as jnp
from jax import lax
from jax.experimental import pallas as pl
from jax.experimental.pallas import tpu as pltpu
from jax.experimental.pallas import tpu_sc as plsc

NTOKEN = 10000
NINP = 256
NHID = 512
N = 256
R = 4
W = 64
BATCH = 64
SEQ = 20
XI = R * W + R + W + 1 + W + W + R + 1 + 1 + 3 * R

BB = 8
EPS = 1e-6


def _softplus(x):
    return jnp.maximum(x, 0.0) + jnp.log1p(jnp.exp(-jnp.abs(x)))


def _softmax(x):
    m = jnp.max(x, axis=-1, keepdims=True)
    e = jnp.exp(x - m)
    return e / jnp.sum(e, axis=-1, keepdims=True)


def _sc_gather(table, idx):
    info = plsc.get_sparse_core_info()
    nw = info.num_cores * info.num_subcores
    b = idx.shape[0]
    b_per_w = b // nw
    mesh = plsc.VectorSubcoreMesh(core_axis_name="c", subcore_axis_name="s")

    @functools.partial(
        pl.kernel, mesh=mesh,
        out_type=jax.ShapeDtypeStruct((b, NINP), jnp.float32),
        scratch_types=[
            pltpu.VMEM((b_per_w,), jnp.int32),
            pltpu.VMEM((b_per_w, NINP), jnp.float32),
            pltpu.SemaphoreType.DMA,
        ],
    )
    def k(table_hbm, idx_hbm, out_hbm, idx_v, rows_v, sem):
        wid = lax.axis_index("s") * info.num_cores + lax.axis_index("c")
        base = wid * b_per_w
        pltpu.sync_copy(idx_hbm.at[pl.ds(base, b_per_w)], idx_v)
        pltpu.async_copy(table_hbm.at[idx_v], rows_v, sem).wait()
        pltpu.sync_copy(rows_v, out_hbm.at[pl.ds(base, b_per_w)])

    return k(table, idx)


def _scan_kernel(emb_ref, h0_ref, W_rnn_ref, b_h_ref, W_xi_ref,
                 b_xi_ref, W_out_ref, b_out_ref, outs_ref,
                 memT_ref, link_ref, prec_ref, usage_ref, ww_ref, rw_ref,
                 rv_ref, h_ref):
    f32 = jnp.float32
    dot = functools.partial(jnp.dot, preferred_element_type=f32)
    h_ref[...] = h0_ref[...]
    memT_ref[...] = jnp.full((BB, W, N), 1e-6, f32)
    link_ref[...] = jnp.zeros((BB, N, N), f32)
    prec_ref[...] = jnp.zeros((BB, N), f32)
    usage_ref[...] = jnp.zeros((BB, N), f32)
    ww_ref[...] = jnp.zeros((BB, N), f32)
    rw_ref[...] = jnp.zeros((BB, R, N), f32)
    rv_ref[...] = jnp.zeros((BB, R * W), f32)

    ii = lax.broadcasted_iota(jnp.int32, (N, N), 0)
    jj = lax.broadcasted_iota(jnp.int32, (N, N), 1)
    offdiag = jnp.where(ii == jj, 0.0, 1.0)
    tie_lt = jj < ii
    ones_col = jnp.ones((N, 1), f32)

    def step(t, _):
        x_t = emb_ref[t]
        rw_old = rw_ref[...]
        usage = usage_ref[...]
        ww_prev = ww_ref[...]
        prec = prec_ref[...]

        cin = jnp.concatenate([x_t, rv_ref[...], h_ref[...]], axis=1)
        h = jnp.tanh(dot(cin, W_rnn_ref[...]) + b_h_ref[...])
        xi = dot(h, W_xi_ref[...]) + b_xi_ref[...]

        o = 0
        read_keys = xi[:, o:o + R * W]; o += R * W
        read_str = 1.0 + _softplus(xi[:, o:o + R]); o += R
        write_key = xi[:, o:o + W]; o += W
        write_str = 1.0 + _softplus(xi[:, o:o + 1]); o += 1
        erase = jax.nn.sigmoid(xi[:, o:o + W]); o += W
        write_vec = xi[:, o:o + W]; o += W
        free = jax.nn.sigmoid(xi[:, o:o + R]); o += R
        alloc_gate = jax.nn.sigmoid(xi[:, o:o + 1]); o += 1
        write_gate = jax.nn.sigmoid(xi[:, o:o + 1]); o += 1
        modes_raw = xi[:, o:o + 3 * R]

        ret = jnp.ones((BB, N), f32)
        for rr in range(R):
            ret = ret * (1.0 - free[:, rr:rr + 1] * rw_old[:, rr, :])
        usage = (usage + ww_prev - usage * ww_prev) * ret
        usageT = usage.T
        log_u = jnp.log(jnp.maximum(usage, 1e-30))

        alloc_cols = []
        cw_rows = []
        for b in range(BB):
            u_row = usage[b:b + 1, :]
            ui_col = usageT[:, b:b + 1]
            before = jnp.logical_or(
                u_row < ui_col,
                jnp.logical_and(u_row == ui_col, tie_lt))
            masked = jnp.where(before, log_u[b:b + 1, :], 0.0)
            cplog = dot(masked, ones_col)
            alloc_cols.append((1.0 - ui_col) * jnp.exp(cplog))

            memT_b = memT_ref[b]
            wkey = write_key[b:b + 1, :]
            wdot = dot(wkey, memT_b)
            mem_norm = jnp.sqrt(
                jnp.sum(memT_b * memT_b, axis=0, keepdims=True))
            wk_norm = jnp.sqrt(
                jnp.sum(wkey * wkey, axis=1, keepdims=True))
            cw_rows.append(_softmax(
                write_str[b:b + 1, :] * (wdot / (wk_norm * mem_norm + EPS))))
        allocT = jnp.concatenate(alloc_cols, axis=1)
        alloc = allocT.T
        cw = jnp.concatenate(cw_rows, axis=0)

        ww = write_gate * (alloc_gate * alloc + (1.0 - alloc_gate) * cw)
        wwT = ww.T
        eraseT = erase.T
        wvT = write_vec.T

        rv_rows = []
        out_rw = []
        for b in range(BB):
            ww_row = ww[b:b + 1, :]
            ww_col = wwT[:, b:b + 1]
            memT_b = memT_ref[b]
            memT_b = (memT_b * (1.0 - eraseT[:, b:b + 1] * ww_row)
                      + wvT[:, b:b + 1] * ww_row)
            memT_ref[b] = memT_b

            link_b = link_ref[b]
            link_b = ((1.0 - ww_col - ww_row) * link_b
                      + ww_col * prec[b:b + 1, :])
            link_b = link_b * offdiag
            link_ref[b] = link_b

            rw_b = rw_old[b]
            bwd = dot(rw_b, link_b)
            fwd = dot(link_b, rw_b.T).T

            rkeys = jnp.concatenate(
                [read_keys[b:b + 1, W * rr:W * rr + W] for rr in range(R)],
                axis=0)
            kdot = dot(rkeys, memT_b)
            kn = jnp.sqrt(jnp.sum(rkeys * rkeys, axis=1, keepdims=True))
            mem_norm = jnp.sqrt(
                jnp.sum(memT_b * memT_b, axis=0, keepdims=True))
            rstr_col = jnp.concatenate(
                [read_str[b:b + 1, rr:rr + 1] for rr in range(R)], axis=0)
            cr = _softmax(rstr_col * (kdot / (kn * mem_norm + EPS)))

            rw_rows = []
            for rr in range(R):
                m = _softmax(modes_raw[b:b + 1, 3 * rr:3 * rr + 3])
                rw_rows.append(m[:, 0:1] * bwd[rr:rr + 1, :]
                               + m[:, 1:2] * cr[rr:rr + 1, :]
                               + m[:, 2:3] * fwd[rr:rr + 1, :])
            rw_new_b = jnp.concatenate(rw_rows, axis=0)
            out_rw.append(rw_new_b[None])
            rvT_b = dot(memT_b, rw_new_b.T)
            rv_rows.append(jnp.concatenate(
                [rvT_b[:, rr:rr + 1].T for rr in range(R)], axis=1))
        rv_flat = jnp.concatenate(rv_rows, axis=0)

        prec = (1.0 - jnp.sum(ww, axis=1, keepdims=True)) * prec + ww

        out = dot(jnp.concatenate([h, rv_flat], axis=1),
                  W_out_ref[...]) + b_out_ref[...]
        outs_ref[t] = out

        prec_ref[...] = prec
        usage_ref[...] = usage
        ww_ref[...] = ww
        rw_ref[...] = jnp.concatenate(out_rw, axis=0)
        rv_ref[...] = rv_flat
        h_ref[...] = h
        return 0

    lax.fori_loop(0, SEQ, step, 0)


def _run_scan(emb, h0, W_rnn, b_h, W_xi, b_xi, W_out, b_out):
    f32 = jnp.float32
    nb = BATCH // BB
    full = lambda i: (0, 0)
    return pl.pallas_call(
        _scan_kernel,
        grid=(nb,),
        in_specs=[
            pl.BlockSpec((SEQ, BB, NINP), lambda i: (0, i, 0)),
            pl.BlockSpec((BB, NHID), lambda i: (i, 0)),
            pl.BlockSpec((NINP + R * W + NHID, NHID), full),
            pl.BlockSpec((1, NHID), full),
            pl.BlockSpec((NHID, XI), full),
            pl.BlockSpec((1, XI), full),
            pl.BlockSpec((NHID + R * W, NINP), full),
            pl.BlockSpec((1, NINP), full),
        ],
        out_specs=pl.BlockSpec((SEQ, BB, NINP), lambda i: (0, i, 0)),
        out_shape=jax.ShapeDtypeStruct((SEQ, BATCH, NINP), f32),
        scratch_shapes=[
            pltpu.VMEM((BB, W, N), f32),
            pltpu.VMEM((BB, N, N), f32),
            pltpu.VMEM((BB, N), f32),
            pltpu.VMEM((BB, N), f32),
            pltpu.VMEM((BB, N), f32),
            pltpu.VMEM((BB, R, N), f32),
            pltpu.VMEM((BB, R * W), f32),
            pltpu.VMEM((BB, NHID), f32),
        ],
    )(emb, h0, W_rnn, b_h, W_xi, b_xi, W_out, b_out)


def _decode_kernel(x_ref, w_ref, b_ref, o_ref):
    o_ref[...] = (jnp.dot(x_ref[...], w_ref[...],
                          preferred_element_type=jnp.float32) + b_ref[...])


def _run_decode(x2d, W_dec, b_dec):
    cb = 1280
    nb = pl.cdiv(NTOKEN, cb)
    rows = x2d.shape[0]
    return pl.pallas_call(
        _decode_kernel,
        grid=(nb,),
        in_specs=[
            pl.BlockSpec((rows, NINP), lambda j: (0, 0)),
            pl.BlockSpec((NINP, cb), lambda j: (0, j)),
            pl.BlockSpec((1, cb), lambda j: (0, j)),
        ],
        out_specs=pl.BlockSpec((rows, cb), lambda j: (0, j)),
        out_shape=jax.ShapeDtypeStruct((rows, NTOKEN), jnp.float32),
    )(x2d, W_dec, b_dec)


def kernel(input, hidden, encoder_w, W_ih, W_hh, b_h, W_xi, b_xi, W_out,
           b_out, W_dec, b_dec):
    idx_tm = jnp.transpose(input).reshape(SEQ * BATCH)
    emb2d = _sc_gather(encoder_w, idx_tm)
    emb = emb2d.reshape(SEQ, BATCH, NINP)

    W_rnn = jnp.concatenate([W_ih, W_hh], axis=0)
    outs = _run_scan(
        emb, hidden[0], W_rnn, b_h.reshape(1, NHID), W_xi,
        b_xi.reshape(1, XI), W_out, b_out.reshape(1, NINP))

    x2d = jnp.transpose(outs, (1, 0, 2)).reshape(BATCH * SEQ, NINP)
    decoded = _run_decode(x2d, W_dec, b_dec.reshape(1, NTOKEN))
    return decoded.reshape(BATCH, SEQ, NTOKEN)


# speedup vs baseline: 1.1489x; 1.0007x over previous
import functools

import jax
import jax.numpy as jnp
from jax import lax
from jax.experimental import pallas as pl
from jax.experimental.pallas import tpu as pltpu
from jax.experimental.pallas import tpu_sc as plsc

NTOKEN = 10000
NINP = 256
NHID = 512
N = 256
R = 4
W = 64
BATCH = 64
SEQ = 20
XI = R * W + R + W + 1 + W + W + R + 1 + 1 + 3 * R

BB = 8
EPS = 1e-6


def _softplus(x):
    return jnp.maximum(x, 0.0) + jnp.log1p(jnp.exp(-jnp.abs(x)))


def _softmax(x):
    m = jnp.max(x, axis=-1, keepdims=True)
    e = jnp.exp(x - m)
    return e / jnp.sum(e, axis=-1, keepdims=True)


def _sc_gather(table, idx):
    info = plsc.get_sparse_core_info()
    nw = info.num_cores * info.num_subcores
    b = idx.shape[0]
    b_per_w = b // nw
    mesh = plsc.VectorSubcoreMesh(core_axis_name="c", subcore_axis_name="s")

    @functools.partial(
        pl.kernel, mesh=mesh,
        out_type=jax.ShapeDtypeStruct((b, NINP), jnp.float32),
        scratch_types=[
            pltpu.VMEM((b_per_w,), jnp.int32),
            pltpu.VMEM((b_per_w, NINP), jnp.float32),
            pltpu.SemaphoreType.DMA,
        ],
    )
    def k(table_hbm, idx_hbm, out_hbm, idx_v, rows_v, sem):
        wid = lax.axis_index("s") * info.num_cores + lax.axis_index("c")
        base = wid * b_per_w
        pltpu.sync_copy(idx_hbm.at[pl.ds(base, b_per_w)], idx_v)
        pltpu.async_copy(table_hbm.at[idx_v], rows_v, sem).wait()
        pltpu.sync_copy(rows_v, out_hbm.at[pl.ds(base, b_per_w)])

    return k(table, idx)


def _scan_kernel(emb_ref, h0_ref, W_rnn_ref, b_h_ref, W_xi_ref,
                 b_xi_ref, W_out_ref, b_out_ref, outs_ref, *scratch):
    memT_refs = scratch[0:BB]
    link_refs = scratch[BB:2 * BB]
    prec_ref, usage_ref, ww_ref, rw_ref, rv_ref, h_ref = scratch[2 * BB:]
    f32 = jnp.float32
    dot = functools.partial(jnp.dot, preferred_element_type=f32)
    h_ref[...] = h0_ref[...]
    for b in range(BB):
        memT_refs[b][...] = jnp.full((W, N), 1e-6, f32)
        link_refs[b][...] = jnp.zeros((N, N), f32)
    prec_ref[...] = jnp.zeros((BB, N), f32)
    usage_ref[...] = jnp.zeros((BB, N), f32)
    ww_ref[...] = jnp.zeros((BB, N), f32)
    rw_ref[...] = jnp.zeros((BB, R, N), f32)
    rv_ref[...] = jnp.zeros((BB, R * W), f32)

    ii = lax.broadcasted_iota(jnp.int32, (N, N), 0)
    jj = lax.broadcasted_iota(jnp.int32, (N, N), 1)
    offdiag = jnp.where(ii == jj, 0.0, 1.0)
    tie_lt = jj < ii
    ones_col = jnp.ones((N, 1), f32)

    def step(t, _):
        x_t = emb_ref[t]
        rw_old = rw_ref[...]
        usage = usage_ref[...]
        ww_prev = ww_ref[...]
        prec = prec_ref[...]

        cin = jnp.concatenate([x_t, rv_ref[...], h_ref[...]], axis=1)
        h = jnp.tanh(dot(cin, W_rnn_ref[...]) + b_h_ref[...])
        xi = dot(h, W_xi_ref[...]) + b_xi_ref[...]

        o = 0
        read_keys = xi[:, o:o + R * W]; o += R * W
        read_str = 1.0 + _softplus(xi[:, o:o + R]); o += R
        write_key = xi[:, o:o + W]; o += W
        write_str = 1.0 + _softplus(xi[:, o:o + 1]); o += 1
        erase = jax.nn.sigmoid(xi[:, o:o + W]); o += W
        write_vec = xi[:, o:o + W]; o += W
        free = jax.nn.sigmoid(xi[:, o:o + R]); o += R
        alloc_gate = jax.nn.sigmoid(xi[:, o:o + 1]); o += 1
        write_gate = jax.nn.sigmoid(xi[:, o:o + 1]); o += 1
        modes_raw = xi[:, o:o + 3 * R]

        ret = jnp.ones((BB, N), f32)
        for rr in range(R):
            ret = ret * (1.0 - free[:, rr:rr + 1] * rw_old[:, rr, :])
        usage = (usage + ww_prev - usage * ww_prev) * ret
        usageT = usage.T
        log_u = jnp.log(jnp.maximum(usage, 1e-30))

        alloc_cols = []
        cw_rows = []
        for b in range(BB):
            u_row = usage[b:b + 1, :]
            ui_col = usageT[:, b:b + 1]
            before = jnp.logical_or(
                u_row < ui_col,
                jnp.logical_and(u_row == ui_col, tie_lt))
            masked = jnp.where(before, log_u[b:b + 1, :], 0.0)
            cplog = dot(masked, ones_col)
            alloc_cols.append((1.0 - ui_col) * jnp.exp(cplog))

            memT_b = memT_refs[b][...]
            wkey = write_key[b:b + 1, :]
            wdot = dot(wkey, memT_b)
            mem_norm = jnp.sqrt(
                jnp.sum(memT_b * memT_b, axis=0, keepdims=True))
            wk_norm = jnp.sqrt(
                jnp.sum(wkey * wkey, axis=1, keepdims=True))
            cw_rows.append(_softmax(
                write_str[b:b + 1, :] * (wdot / (wk_norm * mem_norm + EPS))))
        allocT = jnp.concatenate(alloc_cols, axis=1)
        alloc = allocT.T
        cw = jnp.concatenate(cw_rows, axis=0)

        ww = write_gate * (alloc_gate * alloc + (1.0 - alloc_gate) * cw)
        wwT = ww.T
        eraseT = erase.T
        wvT = write_vec.T

        rv_rows = []
        out_rw = []
        for b in range(BB):
            ww_row = ww[b:b + 1, :]
            ww_col = wwT[:, b:b + 1]
            memT_b = memT_refs[b][...]
            memT_b = (memT_b * (1.0 - eraseT[:, b:b + 1] * ww_row)
                      + wvT[:, b:b + 1] * ww_row)
            memT_refs[b][...] = memT_b

            link_b = link_refs[b][...]
            link_b = ((1.0 - ww_col - ww_row) * link_b
                      + ww_col * prec[b:b + 1, :])
            link_b = link_b * offdiag
            link_refs[b][...] = link_b

            rw_b = rw_old[b]
            bwd = dot(rw_b, link_b)
            fwd = dot(link_b, rw_b.T).T

            rkeys = jnp.concatenate(
                [read_keys[b:b + 1, W * rr:W * rr + W] for rr in range(R)],
                axis=0)
            kdot = dot(rkeys, memT_b)
            kn = jnp.sqrt(jnp.sum(rkeys * rkeys, axis=1, keepdims=True))
            mem_norm = jnp.sqrt(
                jnp.sum(memT_b * memT_b, axis=0, keepdims=True))
            rstr_col = jnp.concatenate(
                [read_str[b:b + 1, rr:rr + 1] for rr in range(R)], axis=0)
            cr = _softmax(rstr_col * (kdot / (kn * mem_norm + EPS)))

            rw_rows = []
            for rr in range(R):
                m = _softmax(modes_raw[b:b + 1, 3 * rr:3 * rr + 3])
                rw_rows.append(m[:, 0:1] * bwd[rr:rr + 1, :]
                               + m[:, 1:2] * cr[rr:rr + 1, :]
                               + m[:, 2:3] * fwd[rr:rr + 1, :])
            rw_new_b = jnp.concatenate(rw_rows, axis=0)
            out_rw.append(rw_new_b[None])
            rvT_b = dot(memT_b, rw_new_b.T)
            rv_rows.append(jnp.concatenate(
                [rvT_b[:, rr:rr + 1].T for rr in range(R)], axis=1))
        rv_flat = jnp.concatenate(rv_rows, axis=0)

        prec = (1.0 - jnp.sum(ww, axis=1, keepdims=True)) * prec + ww

        out = dot(jnp.concatenate([h, rv_flat], axis=1),
                  W_out_ref[...]) + b_out_ref[...]
        outs_ref[t] = out

        prec_ref[...] = prec
        usage_ref[...] = usage
        ww_ref[...] = ww
        rw_ref[...] = jnp.concatenate(out_rw, axis=0)
        rv_ref[...] = rv_flat
        h_ref[...] = h
        return 0

    lax.fori_loop(0, SEQ, step, 0)


def _run_scan(emb, h0, W_rnn, b_h, W_xi, b_xi, W_out, b_out):
    f32 = jnp.float32
    nb = BATCH // BB
    full = lambda i: (0, 0)
    return pl.pallas_call(
        _scan_kernel,
        grid=(nb,),
        in_specs=[
            pl.BlockSpec((SEQ, BB, NINP), lambda i: (0, i, 0)),
            pl.BlockSpec((BB, NHID), lambda i: (i, 0)),
            pl.BlockSpec((NINP + R * W + NHID, NHID), full),
            pl.BlockSpec((1, NHID), full),
            pl.BlockSpec((NHID, XI), full),
            pl.BlockSpec((1, XI), full),
            pl.BlockSpec((NHID + R * W, NINP), full),
            pl.BlockSpec((1, NINP), full),
        ],
        out_specs=pl.BlockSpec((SEQ, BB, NINP), lambda i: (0, i, 0)),
        out_shape=jax.ShapeDtypeStruct((SEQ, BATCH, NINP), f32),
        scratch_shapes=(
            [pltpu.VMEM((W, N), f32) for _ in range(BB)]
            + [pltpu.VMEM((N, N), f32) for _ in range(BB)]
            + [
                pltpu.VMEM((BB, N), f32),
                pltpu.VMEM((BB, N), f32),
                pltpu.VMEM((BB, N), f32),
                pltpu.VMEM((BB, R, N), f32),
                pltpu.VMEM((BB, R * W), f32),
                pltpu.VMEM((BB, NHID), f32),
            ]
        ),
    )(emb, h0, W_rnn, b_h, W_xi, b_xi, W_out, b_out)


def _decode_kernel(x_ref, w_ref, b_ref, o_ref):
    o_ref[...] = (jnp.dot(x_ref[...], w_ref[...],
                          preferred_element_type=jnp.float32) + b_ref[...])


def _run_decode(x2d, W_dec, b_dec):
    cb = 1280
    nb = pl.cdiv(NTOKEN, cb)
    rows = x2d.shape[0]
    return pl.pallas_call(
        _decode_kernel,
        grid=(nb,),
        in_specs=[
            pl.BlockSpec((rows, NINP), lambda j: (0, 0)),
            pl.BlockSpec((NINP, cb), lambda j: (0, j)),
            pl.BlockSpec((1, cb), lambda j: (0, j)),
        ],
        out_specs=pl.BlockSpec((rows, cb), lambda j: (0, j)),
        out_shape=jax.ShapeDtypeStruct((rows, NTOKEN), jnp.float32),
    )(x2d, W_dec, b_dec)


def kernel(input, hidden, encoder_w, W_ih, W_hh, b_h, W_xi, b_xi, W_out,
           b_out, W_dec, b_dec):
    idx_tm = jnp.transpose(input).reshape(SEQ * BATCH)
    emb2d = _sc_gather(encoder_w, idx_tm)
    emb = emb2d.reshape(SEQ, BATCH, NINP)

    W_rnn = jnp.concatenate([W_ih, W_hh], axis=0)
    outs = _run_scan(
        emb, hidden[0], W_rnn, b_h.reshape(1, NHID), W_xi,
        b_xi.reshape(1, XI), W_out, b_out.reshape(1, NINP))

    x2d = jnp.transpose(outs, (1, 0, 2)).reshape(BATCH * SEQ, NINP)
    decoded = _run_decode(x2d, W_dec, b_dec.reshape(1, NTOKEN))
    return decoded.reshape(BATCH, SEQ, NTOKEN)


# speedup vs baseline: 1.8880x; 1.6434x over previous
import functools

import jax
import jax.numpy as jnp
from jax import lax
from jax.experimental import pallas as pl
from jax.experimental.pallas import tpu as pltpu
from jax.experimental.pallas import tpu_sc as plsc

NTOKEN = 10000
NINP = 256
NHID = 512
N = 256
R = 4
W = 64
BATCH = 64
SEQ = 20
XI = R * W + R + W + 1 + W + W + R + 1 + 1 + 3 * R

BB = 8
EPS = 1e-6


def _softplus(x):
    return jnp.maximum(x, 0.0) + jnp.log1p(jnp.exp(-jnp.abs(x)))


def _sc_gather(table, idx):
    info = plsc.get_sparse_core_info()
    nw = info.num_cores * info.num_subcores
    b = idx.shape[0]
    b_per_w = b // nw
    mesh = plsc.VectorSubcoreMesh(core_axis_name="c", subcore_axis_name="s")

    @functools.partial(
        pl.kernel, mesh=mesh,
        out_type=jax.ShapeDtypeStruct((b, NINP), jnp.float32),
        scratch_types=[
            pltpu.VMEM((b_per_w,), jnp.int32),
            pltpu.VMEM((b_per_w, NINP), jnp.float32),
            pltpu.SemaphoreType.DMA,
        ],
    )
    def k(table_hbm, idx_hbm, out_hbm, idx_v, rows_v, sem):
        wid = lax.axis_index("s") * info.num_cores + lax.axis_index("c")
        base = wid * b_per_w
        pltpu.sync_copy(idx_hbm.at[pl.ds(base, b_per_w)], idx_v)
        pltpu.async_copy(table_hbm.at[idx_v], rows_v, sem).wait()
        pltpu.sync_copy(rows_v, out_hbm.at[pl.ds(base, b_per_w)])

    return k(table, idx)


def _scan_kernel(emb_ref, h0_ref, W_ih_ref, W_hh_ref, b_h_ref, W_xi_ref,
                 b_xi_ref, W_out_ref, b_out_ref, outs_ref, *scratch):
    mem_refs = scratch[0:BB]
    link_refs = scratch[BB:2 * BB]
    (prec_ref, usage_ref, ww_ref, rw_ref, rv_ref, h_ref) = scratch[2 * BB:]
    f32 = jnp.float32
    dot = functools.partial(jnp.dot, preferred_element_type=f32)
    dgt = lambda a, b: lax.dot_general(
        a, b, (((1,), (1,)), ((), ())), preferred_element_type=f32)

    h_ref[...] = h0_ref[...]
    for b in range(BB):
        mem_refs[b][...] = jnp.full((N, W), 1e-6, f32)
        link_refs[b][...] = jnp.zeros((N, N), f32)
    prec_ref[...] = jnp.zeros((BB, N), f32)
    usage_ref[...] = jnp.zeros((BB, N), f32)
    ww_ref[...] = jnp.zeros((BB, N), f32)
    rw_ref[...] = jnp.zeros((R, BB, N), f32)
    rv_ref[...] = jnp.zeros((R, BB, W), f32)

    ii = lax.broadcasted_iota(jnp.int32, (N, N), 0)
    jj = lax.broadcasted_iota(jnp.int32, (N, N), 1)
    offdiag = jnp.where(ii == jj, 0.0, 1.0)
    tie_lt = jj < ii
    ones_row = jnp.ones((1, N), f32)
    ones_col64 = jnp.ones((W, 1), f32)
    bcol = lambda c: dot(c, ones_row)

    def step(t, _):
        x_t = emb_ref[t]
        usage = usage_ref[...]
        ww_prev = ww_ref[...]
        prec = prec_ref[...]
        rw_old = [rw_ref[r] for r in range(R)]
        mem_old = [mem_refs[b][...] for b in range(BB)]

        pre = dot(x_t, W_ih_ref[0:NINP, :]) + dot(h_ref[...], W_hh_ref[...])
        for r in range(R):
            pre = pre + dot(rv_ref[r], W_ih_ref[NINP + W * r:NINP + W * (r + 1), :])
        h = jnp.tanh(pre + b_h_ref[...])
        xi = dot(h, W_xi_ref[...]) + b_xi_ref[...]

        o = 0
        read_keys = xi[:, o:o + R * W]; o += R * W
        read_str = 1.0 + _softplus(xi[:, o:o + R]); o += R
        write_key = xi[:, o:o + W]; o += W
        write_str = 1.0 + _softplus(xi[:, o:o + 1]); o += 1
        erase = jax.nn.sigmoid(xi[:, o:o + W]); o += W
        write_vec = xi[:, o:o + W]; o += W
        free = jax.nn.sigmoid(xi[:, o:o + R]); o += R
        alloc_gate = jax.nn.sigmoid(xi[:, o:o + 1]); o += 1
        write_gate = jax.nn.sigmoid(xi[:, o:o + 1]); o += 1
        modes_raw = xi[:, o:o + 3 * R]

        ret = 1.0 - bcol(free[:, 0:1]) * rw_old[0]
        for r in range(1, R):
            ret = ret * (1.0 - bcol(free[:, r:r + 1]) * rw_old[r])
        usage = (usage + ww_prev - usage * ww_prev) * ret
        usageT = usage.T
        log_u = jnp.log(jnp.maximum(usage, 1e-30))

        alloc_cols, nrm2_cols, wdot_rows = [], [], []
        for b in range(BB):
            ui_col = usageT[:, b:b + 1]
            ui_mat = bcol(ui_col)
            u_row = usage[b:b + 1, :]
            before = jnp.logical_or(
                u_row < ui_mat,
                jnp.logical_and(u_row == ui_mat, tie_lt))
            masked = jnp.where(before, log_u[b:b + 1, :], 0.0)
            cplog = dot(masked, ones_row.T)
            alloc_cols.append((1.0 - ui_col) * jnp.exp(cplog))
            mem_b = mem_old[b]
            nrm2_cols.append(dot(mem_b * mem_b, ones_col64))
            wdot_rows.append(dgt(write_key[b:b + 1, :], mem_b))
        colsT = jnp.concatenate(alloc_cols + nrm2_cols, axis=1)
        cols = colsT.T
        alloc = cols[0:BB, :]
        mem_norm_old = jnp.sqrt(cols[BB:2 * BB, :])
        wdot = jnp.concatenate(wdot_rows, axis=0)

        wk_norm = jnp.sqrt(jnp.sum(write_key * write_key, axis=1,
                                   keepdims=True))
        score = bcol(write_str) * wdot / (bcol(wk_norm) * mem_norm_old + EPS)
        m_ = jnp.max(score, axis=1, keepdims=True)
        e_ = jnp.exp(score - bcol(m_))
        cw = e_ * bcol(1.0 / jnp.sum(e_, axis=1, keepdims=True))

        ag = bcol(alloc_gate)
        ww = bcol(write_gate) * (ag * alloc + (1.0 - ag) * cw)
        wwT = ww.T

        mem_new = []
        nrm2n_cols, kdot_rows, bwd_rows, fwd_rows = [], [], [], []
        for b in range(BB):
            ww_col = wwT[:, b:b + 1]
            ww_row = ww[b:b + 1, :]
            e_mat = dot(ww_col, erase[b:b + 1, :])
            wv_mat = dot(ww_col, write_vec[b:b + 1, :])
            mem_b = mem_old[b]
            mem_b = mem_b - mem_b * e_mat + wv_mat
            mem_refs[b][...] = mem_b
            mem_new.append(mem_b)
            nrm2n_cols.append(dot(mem_b * mem_b, ones_col64))

            link_b = link_refs[b][...]
            link_b = ((1.0 - bcol(ww_col) - ww_row) * link_b
                      + dot(ww_col, prec[b:b + 1, :]))
            link_b = link_b * offdiag
            link_refs[b][...] = link_b

            rw_b = jnp.concatenate(
                [rw_old[r][b:b + 1, :] for r in range(R)], axis=0)
            bwd_rows.append(dot(rw_b, link_b))
            fwd_rows.append(dgt(rw_b, link_b))
            rkeys = jnp.concatenate(
                [read_keys[b:b + 1, W * r:W * (r + 1)] for r in range(R)],
                axis=0)
            kdot_rows.append(dgt(rkeys, mem_b))
        nrm2nT = jnp.concatenate(nrm2n_cols, axis=1)
        mem_norm_new = jnp.sqrt(nrm2nT.T)

        rw_new = []
        for r in range(R):
            bwd_r = jnp.concatenate(
                [bwd_rows[b][r:r + 1, :] for b in range(BB)], axis=0)
            fwd_r = jnp.concatenate(
                [fwd_rows[b][r:r + 1, :] for b in range(BB)], axis=0)
            kdot_r = jnp.concatenate(
                [kdot_rows[b][r:r + 1, :] for b in range(BB)], axis=0)
            rk = read_keys[:, W * r:W * (r + 1)]
            kn = jnp.sqrt(jnp.sum(rk * rk, axis=1, keepdims=True))
            sc_ = (bcol(read_str[:, r:r + 1]) * kdot_r
                   / (bcol(kn) * mem_norm_new + EPS))
            m2 = jnp.max(sc_, axis=1, keepdims=True)
            e2 = jnp.exp(sc_ - bcol(m2))
            cr_r = e2 * bcol(1.0 / jnp.sum(e2, axis=1, keepdims=True))

            msl = modes_raw[:, 3 * r:3 * r + 3]
            mm = jnp.max(msl, axis=1, keepdims=True)
            me = jnp.exp(msl - mm)
            ms = me / jnp.sum(me, axis=1, keepdims=True)
            rw_r = (bcol(ms[:, 0:1]) * bwd_r + bcol(ms[:, 1:2]) * cr_r
                    + bcol(ms[:, 2:3]) * fwd_r)
            rw_new.append(rw_r)
            rw_ref[r] = rw_r

        rv_heads = [[] for _ in range(R)]
        for b in range(BB):
            rw_nb = jnp.concatenate(
                [rw_new[r][b:b + 1, :] for r in range(R)], axis=0)
            rv4 = dot(rw_nb, mem_new[b])
            for r in range(R):
                rv_heads[r].append(rv4[r:r + 1, :])
        out = dot(h, W_out_ref[0:NHID, :])
        for r in range(R):
            rv_r = jnp.concatenate(rv_heads[r], axis=0)
            rv_ref[r] = rv_r
            out = out + dot(rv_r, W_out_ref[NHID + W * r:NHID + W * (r + 1), :])
        outs_ref[t] = out + b_out_ref[...]

        prec_ref[...] = (1.0 - jnp.sum(ww, axis=1, keepdims=True)) * prec + ww
        usage_ref[...] = usage
        ww_ref[...] = ww
        h_ref[...] = h
        return 0

    lax.fori_loop(0, SEQ, step, 0)


def _run_scan(emb, h0, W_ih, W_hh, b_h, W_xi, b_xi, W_out, b_out):
    f32 = jnp.float32
    nb = BATCH // BB
    full = lambda i: (0, 0)
    return pl.pallas_call(
        _scan_kernel,
        grid=(nb,),
        in_specs=[
            pl.BlockSpec((SEQ, BB, NINP), lambda i: (0, i, 0)),
            pl.BlockSpec((BB, NHID), lambda i: (i, 0)),
            pl.BlockSpec((NINP + R * W, NHID), full),
            pl.BlockSpec((NHID, NHID), full),
            pl.BlockSpec((1, NHID), full),
            pl.BlockSpec((NHID, XI), full),
            pl.BlockSpec((1, XI), full),
            pl.BlockSpec((NHID + R * W, NINP), full),
            pl.BlockSpec((1, NINP), full),
        ],
        out_specs=pl.BlockSpec((SEQ, BB, NINP), lambda i: (0, i, 0)),
        out_shape=jax.ShapeDtypeStruct((SEQ, BATCH, NINP), f32),
        scratch_shapes=(
            [pltpu.VMEM((N, W), f32) for _ in range(BB)]
            + [pltpu.VMEM((N, N), f32) for _ in range(BB)]
            + [
                pltpu.VMEM((BB, N), f32),
                pltpu.VMEM((BB, N), f32),
                pltpu.VMEM((BB, N), f32),
                pltpu.VMEM((R, BB, N), f32),
                pltpu.VMEM((R, BB, W), f32),
                pltpu.VMEM((BB, NHID), f32),
            ]
        ),
    )(emb, h0, W_ih, W_hh, b_h, W_xi, b_xi, W_out, b_out)


def _decode_kernel(x_ref, w_ref, b_ref, o_ref):
    o_ref[...] = (jnp.dot(x_ref[...], w_ref[...],
                          preferred_element_type=jnp.float32) + b_ref[...])


def _run_decode(x2d, W_dec, b_dec):
    cb = 1280
    nb = pl.cdiv(NTOKEN, cb)
    rows = x2d.shape[0]
    return pl.pallas_call(
        _decode_kernel,
        grid=(nb,),
        in_specs=[
            pl.BlockSpec((rows, NINP), lambda j: (0, 0)),
            pl.BlockSpec((NINP, cb), lambda j: (0, j)),
            pl.BlockSpec((1, cb), lambda j: (0, j)),
        ],
        out_specs=pl.BlockSpec((rows, cb), lambda j: (0, j)),
        out_shape=jax.ShapeDtypeStruct((rows, NTOKEN), jnp.float32),
    )(x2d, W_dec, b_dec)


def kernel(input, hidden, encoder_w, W_ih, W_hh, b_h, W_xi, b_xi, W_out,
           b_out, W_dec, b_dec):
    idx_tm = jnp.transpose(input).reshape(SEQ * BATCH)
    emb2d = _sc_gather(encoder_w, idx_tm)
    emb = emb2d.reshape(SEQ, BATCH, NINP)

    outs = _run_scan(
        emb, hidden[0], W_ih, W_hh, b_h.reshape(1, NHID), W_xi,
        b_xi.reshape(1, XI), W_out, b_out.reshape(1, NINP))

    x2d = jnp.transpose(outs, (1, 0, 2)).reshape(BATCH * SEQ, NINP)
    decoded = _run_decode(x2d, W_dec, b_dec.reshape(1, NTOKEN))
    return decoded.reshape(BATCH, SEQ, NTOKEN)


# speedup vs baseline: 2.7102x; 1.4354x over previous
import functools

import jax
import jax.numpy as jnp
from jax import lax
from jax.experimental import pallas as pl
from jax.experimental.pallas import tpu as pltpu
from jax.experimental.pallas import tpu_sc as plsc

NTOKEN = 10000
NINP = 256
NHID = 512
N = 256
R = 4
W = 64
BATCH = 64
SEQ = 20
XI = R * W + R + W + 1 + W + W + R + 1 + 1 + 3 * R

BB = 32
EPS = 1e-6


def _softplus(x):
    return jnp.maximum(x, 0.0) + jnp.log1p(jnp.exp(-jnp.abs(x)))


def _sc_gather(table, idx):
    info = plsc.get_sparse_core_info()
    nw = info.num_cores * info.num_subcores
    b = idx.shape[0]
    b_per_w = b // nw
    mesh = plsc.VectorSubcoreMesh(core_axis_name="c", subcore_axis_name="s")

    @functools.partial(
        pl.kernel, mesh=mesh,
        out_type=jax.ShapeDtypeStruct((b, NINP), jnp.float32),
        scratch_types=[
            pltpu.VMEM((b_per_w,), jnp.int32),
            pltpu.VMEM((b_per_w, NINP), jnp.float32),
            pltpu.SemaphoreType.DMA,
        ],
    )
    def k(table_hbm, idx_hbm, out_hbm, idx_v, rows_v, sem):
        wid = lax.axis_index("s") * info.num_cores + lax.axis_index("c")
        base = wid * b_per_w
        pltpu.sync_copy(idx_hbm.at[pl.ds(base, b_per_w)], idx_v)
        pltpu.async_copy(table_hbm.at[idx_v], rows_v, sem).wait()
        pltpu.sync_copy(rows_v, out_hbm.at[pl.ds(base, b_per_w)])

    return k(table, idx)


def _scan_kernel(emb_ref, h0_ref, W_ih_ref, W_hh_ref, b_h_ref, W_xi_ref,
                 b_xi_ref, W_out_ref, b_out_ref, outs_ref, *scratch):
    mem_refs = scratch[0:BB]
    link_refs = scratch[BB:2 * BB]
    (prec_ref, usage_ref, ww_ref, rw_ref, rv_ref, h_ref) = scratch[2 * BB:]
    f32 = jnp.float32
    dot = functools.partial(jnp.dot, preferred_element_type=f32)
    dgt = lambda a, b: lax.dot_general(
        a, b, (((1,), (1,)), ((), ())), preferred_element_type=f32)

    h_ref[...] = h0_ref[...]
    for b in range(BB):
        mem_refs[b][...] = jnp.full((N, W), 1e-6, f32)
        link_refs[b][...] = jnp.zeros((N, N), f32)
    prec_ref[...] = jnp.zeros((BB, N), f32)
    usage_ref[...] = jnp.zeros((BB, N), f32)
    ww_ref[...] = jnp.zeros((BB, N), f32)
    rw_ref[...] = jnp.zeros((R, BB, N), f32)
    rv_ref[...] = jnp.zeros((R, BB, W), f32)

    ii = lax.broadcasted_iota(jnp.int32, (N, N), 0)
    jj = lax.broadcasted_iota(jnp.int32, (N, N), 1)
    offdiag = jnp.where(ii == jj, 0.0, 1.0)
    tie_lt = jj < ii
    ones_row = jnp.ones((1, N), f32)
    ones_col64 = jnp.ones((W, 1), f32)
    bcol = lambda c: dot(c, ones_row)

    def step(t, _):
        x_t = emb_ref[t]
        usage = usage_ref[...]
        ww_prev = ww_ref[...]
        prec = prec_ref[...]
        rw_old = [rw_ref[r] for r in range(R)]
        mem_old = [mem_refs[b][...] for b in range(BB)]

        pre = dot(x_t, W_ih_ref[0:NINP, :]) + dot(h_ref[...], W_hh_ref[...])
        for r in range(R):
            pre = pre + dot(rv_ref[r], W_ih_ref[NINP + W * r:NINP + W * (r + 1), :])
        h = jnp.tanh(pre + b_h_ref[...])
        xi = dot(h, W_xi_ref[...]) + b_xi_ref[...]

        o = 0
        read_keys = xi[:, o:o + R * W]; o += R * W
        read_str = 1.0 + _softplus(xi[:, o:o + R]); o += R
        write_key = xi[:, o:o + W]; o += W
        write_str = 1.0 + _softplus(xi[:, o:o + 1]); o += 1
        erase = jax.nn.sigmoid(xi[:, o:o + W]); o += W
        write_vec = xi[:, o:o + W]; o += W
        free = jax.nn.sigmoid(xi[:, o:o + R]); o += R
        alloc_gate = jax.nn.sigmoid(xi[:, o:o + 1]); o += 1
        write_gate = jax.nn.sigmoid(xi[:, o:o + 1]); o += 1
        modes_raw = xi[:, o:o + 3 * R]

        ret = 1.0 - bcol(free[:, 0:1]) * rw_old[0]
        for r in range(1, R):
            ret = ret * (1.0 - bcol(free[:, r:r + 1]) * rw_old[r])
        usage = (usage + ww_prev - usage * ww_prev) * ret
        usageT = usage.T
        log_u = jnp.log(jnp.maximum(usage, 1e-30))

        alloc_cols, nrm2_cols, wdot_rows = [], [], []
        for b in range(BB):
            ui_col = usageT[:, b:b + 1]
            ui_mat = bcol(ui_col)
            u_row = usage[b:b + 1, :]
            before = jnp.logical_or(
                u_row < ui_mat,
                jnp.logical_and(u_row == ui_mat, tie_lt))
            masked = jnp.where(before, log_u[b:b + 1, :], 0.0)
            cplog = dot(masked, ones_row.T)
            alloc_cols.append((1.0 - ui_col) * jnp.exp(cplog))
            mem_b = mem_old[b]
            nrm2_cols.append(dot(mem_b * mem_b, ones_col64))
            wdot_rows.append(dgt(write_key[b:b + 1, :], mem_b))
        colsT = jnp.concatenate(alloc_cols + nrm2_cols, axis=1)
        cols = colsT.T
        alloc = cols[0:BB, :]
        mem_norm_old = jnp.sqrt(cols[BB:2 * BB, :])
        wdot = jnp.concatenate(wdot_rows, axis=0)

        wk_norm = jnp.sqrt(jnp.sum(write_key * write_key, axis=1,
                                   keepdims=True))
        score = bcol(write_str) * wdot / (bcol(wk_norm) * mem_norm_old + EPS)
        m_ = jnp.max(score, axis=1, keepdims=True)
        e_ = jnp.exp(score - bcol(m_))
        cw = e_ * bcol(1.0 / jnp.sum(e_, axis=1, keepdims=True))

        ag = bcol(alloc_gate)
        ww = bcol(write_gate) * (ag * alloc + (1.0 - ag) * cw)
        wwT = ww.T

        mem_new = []
        nrm2n_cols, kdot_rows, bwd_rows, fwd_rows = [], [], [], []
        for b in range(BB):
            ww_col = wwT[:, b:b + 1]
            ww_row = ww[b:b + 1, :]
            e_mat = dot(ww_col, erase[b:b + 1, :])
            wv_mat = dot(ww_col, write_vec[b:b + 1, :])
            mem_b = mem_old[b]
            mem_b = mem_b - mem_b * e_mat + wv_mat
            mem_refs[b][...] = mem_b
            mem_new.append(mem_b)
            nrm2n_cols.append(dot(mem_b * mem_b, ones_col64))

            link_b = link_refs[b][...]
            link_b = ((1.0 - bcol(ww_col) - ww_row) * link_b
                      + dot(ww_col, prec[b:b + 1, :]))
            link_b = link_b * offdiag
            link_refs[b][...] = link_b

            rw_b = jnp.concatenate(
                [rw_old[r][b:b + 1, :] for r in range(R)], axis=0)
            bwd_rows.append(dot(rw_b, link_b))
            fwd_rows.append(dgt(rw_b, link_b))
            rkeys = jnp.concatenate(
                [read_keys[b:b + 1, W * r:W * (r + 1)] for r in range(R)],
                axis=0)
            kdot_rows.append(dgt(rkeys, mem_b))
        nrm2nT = jnp.concatenate(nrm2n_cols, axis=1)
        mem_norm_new = jnp.sqrt(nrm2nT.T)

        rw_new = []
        for r in range(R):
            bwd_r = jnp.concatenate(
                [bwd_rows[b][r:r + 1, :] for b in range(BB)], axis=0)
            fwd_r = jnp.concatenate(
                [fwd_rows[b][r:r + 1, :] for b in range(BB)], axis=0)
            kdot_r = jnp.concatenate(
                [kdot_rows[b][r:r + 1, :] for b in range(BB)], axis=0)
            rk = read_keys[:, W * r:W * (r + 1)]
            kn = jnp.sqrt(jnp.sum(rk * rk, axis=1, keepdims=True))
            sc_ = (bcol(read_str[:, r:r + 1]) * kdot_r
                   / (bcol(kn) * mem_norm_new + EPS))
            m2 = jnp.max(sc_, axis=1, keepdims=True)
            e2 = jnp.exp(sc_ - bcol(m2))
            cr_r = e2 * bcol(1.0 / jnp.sum(e2, axis=1, keepdims=True))

            msl = modes_raw[:, 3 * r:3 * r + 3]
            mm = jnp.max(msl, axis=1, keepdims=True)
            me = jnp.exp(msl - mm)
            ms = me / jnp.sum(me, axis=1, keepdims=True)
            rw_r = (bcol(ms[:, 0:1]) * bwd_r + bcol(ms[:, 1:2]) * cr_r
                    + bcol(ms[:, 2:3]) * fwd_r)
            rw_new.append(rw_r)
            rw_ref[r] = rw_r

        rv_heads = [[] for _ in range(R)]
        for b in range(BB):
            rw_nb = jnp.concatenate(
                [rw_new[r][b:b + 1, :] for r in range(R)], axis=0)
            rv4 = dot(rw_nb, mem_new[b])
            for r in range(R):
                rv_heads[r].append(rv4[r:r + 1, :])
        out = dot(h, W_out_ref[0:NHID, :])
        for r in range(R):
            rv_r = jnp.concatenate(rv_heads[r], axis=0)
            rv_ref[r] = rv_r
            out = out + dot(rv_r, W_out_ref[NHID + W * r:NHID + W * (r + 1), :])
        outs_ref[t] = out + b_out_ref[...]

        prec_ref[...] = (1.0 - jnp.sum(ww, axis=1, keepdims=True)) * prec + ww
        usage_ref[...] = usage
        ww_ref[...] = ww
        h_ref[...] = h
        return 0

    lax.fori_loop(0, SEQ, step, 0)


def _run_scan(emb, h0, W_ih, W_hh, b_h, W_xi, b_xi, W_out, b_out):
    f32 = jnp.float32
    nb = BATCH // BB
    full = lambda i: (0, 0)
    return pl.pallas_call(
        _scan_kernel,
        grid=(nb,),
        in_specs=[
            pl.BlockSpec((SEQ, BB, NINP), lambda i: (0, i, 0)),
            pl.BlockSpec((BB, NHID), lambda i: (i, 0)),
            pl.BlockSpec((NINP + R * W, NHID), full),
            pl.BlockSpec((NHID, NHID), full),
            pl.BlockSpec((1, NHID), full),
            pl.BlockSpec((NHID, XI), full),
            pl.BlockSpec((1, XI), full),
            pl.BlockSpec((NHID + R * W, NINP), full),
            pl.BlockSpec((1, NINP), full),
        ],
        out_specs=pl.BlockSpec((SEQ, BB, NINP), lambda i: (0, i, 0)),
        out_shape=jax.ShapeDtypeStruct((SEQ, BATCH, NINP), f32),
        scratch_shapes=(
            [pltpu.VMEM((N, W), f32) for _ in range(BB)]
            + [pltpu.VMEM((N, N), f32) for _ in range(BB)]
            + [
                pltpu.VMEM((BB, N), f32),
                pltpu.VMEM((BB, N), f32),
                pltpu.VMEM((BB, N), f32),
                pltpu.VMEM((R, BB, N), f32),
                pltpu.VMEM((R, BB, W), f32),
                pltpu.VMEM((BB, NHID), f32),
            ]
        ),
    )(emb, h0, W_ih, W_hh, b_h, W_xi, b_xi, W_out, b_out)


def _decode_kernel(x_ref, w_ref, b_ref, o_ref):
    o_ref[...] = (jnp.dot(x_ref[...], w_ref[...],
                          preferred_element_type=jnp.float32) + b_ref[...])


def _run_decode(x2d, W_dec, b_dec):
    cb = 1280
    nb = pl.cdiv(NTOKEN, cb)
    rows = x2d.shape[0]
    return pl.pallas_call(
        _decode_kernel,
        grid=(nb,),
        in_specs=[
            pl.BlockSpec((rows, NINP), lambda j: (0, 0)),
            pl.BlockSpec((NINP, cb), lambda j: (0, j)),
            pl.BlockSpec((1, cb), lambda j: (0, j)),
        ],
        out_specs=pl.BlockSpec((rows, cb), lambda j: (0, j)),
        out_shape=jax.ShapeDtypeStruct((rows, NTOKEN), jnp.float32),
    )(x2d, W_dec, b_dec)


def kernel(input, hidden, encoder_w, W_ih, W_hh, b_h, W_xi, b_xi, W_out,
           b_out, W_dec, b_dec):
    idx_tm = jnp.transpose(input).reshape(SEQ * BATCH)
    emb2d = _sc_gather(encoder_w, idx_tm)
    emb = emb2d.reshape(SEQ, BATCH, NINP)

    outs = _run_scan(
        emb, hidden[0], W_ih, W_hh, b_h.reshape(1, NHID), W_xi,
        b_xi.reshape(1, XI), W_out, b_out.reshape(1, NINP))

    x2d = jnp.transpose(outs, (1, 0, 2)).reshape(BATCH * SEQ, NINP)
    decoded = _run_decode(x2d, W_dec, b_dec.reshape(1, NTOKEN))
    return decoded.reshape(BATCH, SEQ, NTOKEN)


# speedup vs baseline: 2.8234x; 1.0418x over previous
import functools

import jax
import jax.numpy as jnp
from jax import lax
from jax.experimental import pallas as pl
from jax.experimental.pallas import tpu as pltpu
from jax.experimental.pallas import tpu_sc as plsc

NTOKEN = 10000
NINP = 256
NHID = 512
N = 256
R = 4
W = 64
BATCH = 64
SEQ = 20
XI = R * W + R + W + 1 + W + W + R + 1 + 1 + 3 * R

BB = 64
EPS = 1e-6


def _softplus(x):
    return jnp.maximum(x, 0.0) + jnp.log1p(jnp.exp(-jnp.abs(x)))


def _sc_gather(table, idx):
    info = plsc.get_sparse_core_info()
    nw = info.num_cores * info.num_subcores
    b = idx.shape[0]
    b_per_w = b // nw
    mesh = plsc.VectorSubcoreMesh(core_axis_name="c", subcore_axis_name="s")

    @functools.partial(
        pl.kernel, mesh=mesh,
        out_type=jax.ShapeDtypeStruct((b, NINP), jnp.float32),
        scratch_types=[
            pltpu.VMEM((b_per_w,), jnp.int32),
            pltpu.VMEM((b_per_w, NINP), jnp.float32),
            pltpu.SemaphoreType.DMA,
        ],
    )
    def k(table_hbm, idx_hbm, out_hbm, idx_v, rows_v, sem):
        wid = lax.axis_index("s") * info.num_cores + lax.axis_index("c")
        base = wid * b_per_w
        pltpu.sync_copy(idx_hbm.at[pl.ds(base, b_per_w)], idx_v)
        pltpu.async_copy(table_hbm.at[idx_v], rows_v, sem).wait()
        pltpu.sync_copy(rows_v, out_hbm.at[pl.ds(base, b_per_w)])

    return k(table, idx)


def _scan_kernel(emb_ref, h0_ref, W_ih_ref, W_hh_ref, b_h_ref, W_xi_ref,
                 b_xi_ref, W_out_ref, b_out_ref, outs_ref, *scratch):
    mem_refs = scratch[0:BB]
    link_refs = scratch[BB:2 * BB]
    (prec_ref, usage_ref, ww_ref, rw_ref, rv_ref, h_ref) = scratch[2 * BB:]
    f32 = jnp.float32
    dot = functools.partial(jnp.dot, preferred_element_type=f32)
    dgt = lambda a, b: lax.dot_general(
        a, b, (((1,), (1,)), ((), ())), preferred_element_type=f32)

    h_ref[...] = h0_ref[...]
    for b in range(BB):
        mem_refs[b][...] = jnp.full((N, W), 1e-6, f32)
        link_refs[b][...] = jnp.zeros((N, N), f32)
    prec_ref[...] = jnp.zeros((BB, N), f32)
    usage_ref[...] = jnp.zeros((BB, N), f32)
    ww_ref[...] = jnp.zeros((BB, N), f32)
    rw_ref[...] = jnp.zeros((R, BB, N), f32)
    rv_ref[...] = jnp.zeros((R, BB, W), f32)

    ii = lax.broadcasted_iota(jnp.int32, (N, N), 0)
    jj = lax.broadcasted_iota(jnp.int32, (N, N), 1)
    offdiag = jnp.where(ii == jj, 0.0, 1.0)
    tie_lt = jj < ii
    ones_row = jnp.ones((1, N), f32)
    ones_col64 = jnp.ones((W, 1), f32)
    bcol = lambda c: dot(c, ones_row)

    def step(t, _):
        x_t = emb_ref[t]
        usage = usage_ref[...]
        ww_prev = ww_ref[...]
        prec = prec_ref[...]
        rw_old = [rw_ref[r] for r in range(R)]
        mem_old = [mem_refs[b][...] for b in range(BB)]

        pre = dot(x_t, W_ih_ref[0:NINP, :]) + dot(h_ref[...], W_hh_ref[...])
        for r in range(R):
            pre = pre + dot(rv_ref[r], W_ih_ref[NINP + W * r:NINP + W * (r + 1), :])
        h = jnp.tanh(pre + b_h_ref[...])
        xi = dot(h, W_xi_ref[...]) + b_xi_ref[...]

        o = 0
        read_keys = xi[:, o:o + R * W]; o += R * W
        read_str = 1.0 + _softplus(xi[:, o:o + R]); o += R
        write_key = xi[:, o:o + W]; o += W
        write_str = 1.0 + _softplus(xi[:, o:o + 1]); o += 1
        erase = jax.nn.sigmoid(xi[:, o:o + W]); o += W
        write_vec = xi[:, o:o + W]; o += W
        free = jax.nn.sigmoid(xi[:, o:o + R]); o += R
        alloc_gate = jax.nn.sigmoid(xi[:, o:o + 1]); o += 1
        write_gate = jax.nn.sigmoid(xi[:, o:o + 1]); o += 1
        modes_raw = xi[:, o:o + 3 * R]

        ret = 1.0 - bcol(free[:, 0:1]) * rw_old[0]
        for r in range(1, R):
            ret = ret * (1.0 - bcol(free[:, r:r + 1]) * rw_old[r])
        usage = (usage + ww_prev - usage * ww_prev) * ret
        usageT = usage.T
        log_u = jnp.log(jnp.maximum(usage, 1e-30))

        alloc_cols, nrm2_cols, wdot_rows = [], [], []
        for b in range(BB):
            ui_col = usageT[:, b:b + 1]
            ui_mat = bcol(ui_col)
            u_row = usage[b:b + 1, :]
            before = jnp.logical_or(
                u_row < ui_mat,
                jnp.logical_and(u_row == ui_mat, tie_lt))
            masked = jnp.where(before, log_u[b:b + 1, :], 0.0)
            cplog = dot(masked, ones_row.T)
            alloc_cols.append((1.0 - ui_col) * jnp.exp(cplog))
            mem_b = mem_old[b]
            nrm2_cols.append(dot(mem_b * mem_b, ones_col64))
            wdot_rows.append(dgt(write_key[b:b + 1, :], mem_b))
        colsT = jnp.concatenate(alloc_cols + nrm2_cols, axis=1)
        cols = colsT.T
        alloc = cols[0:BB, :]
        mem_norm_old = jnp.sqrt(cols[BB:2 * BB, :])
        wdot = jnp.concatenate(wdot_rows, axis=0)

        wk_norm = jnp.sqrt(jnp.sum(write_key * write_key, axis=1,
                                   keepdims=True))
        score = bcol(write_str) * wdot / (bcol(wk_norm) * mem_norm_old + EPS)
        m_ = jnp.max(score, axis=1, keepdims=True)
        e_ = jnp.exp(score - bcol(m_))
        cw = e_ * bcol(1.0 / jnp.sum(e_, axis=1, keepdims=True))

        ag = bcol(alloc_gate)
        ww = bcol(write_gate) * (ag * alloc + (1.0 - ag) * cw)
        wwT = ww.T

        mem_new = []
        nrm2n_cols, kdot_rows, bwd_rows, fwd_rows = [], [], [], []
        for b in range(BB):
            ww_col = wwT[:, b:b + 1]
            ww_row = ww[b:b + 1, :]
            e_mat = dot(ww_col, erase[b:b + 1, :])
            wv_mat = dot(ww_col, write_vec[b:b + 1, :])
            mem_b = mem_old[b]
            mem_b = mem_b - mem_b * e_mat + wv_mat
            mem_refs[b][...] = mem_b
            mem_new.append(mem_b)
            nrm2n_cols.append(dot(mem_b * mem_b, ones_col64))

            link_b = link_refs[b][...]
            link_b = ((1.0 - bcol(ww_col) - ww_row) * link_b
                      + dot(ww_col, prec[b:b + 1, :]))
            link_b = link_b * offdiag
            link_refs[b][...] = link_b

            rw_b = jnp.concatenate(
                [rw_old[r][b:b + 1, :] for r in range(R)], axis=0)
            bwd_rows.append(dot(rw_b, link_b))
            fwd_rows.append(dgt(rw_b, link_b))
            rkeys = jnp.concatenate(
                [read_keys[b:b + 1, W * r:W * (r + 1)] for r in range(R)],
                axis=0)
            kdot_rows.append(dgt(rkeys, mem_b))
        nrm2nT = jnp.concatenate(nrm2n_cols, axis=1)
        mem_norm_new = jnp.sqrt(nrm2nT.T)

        rw_new = []
        for r in range(R):
            bwd_r = jnp.concatenate(
                [bwd_rows[b][r:r + 1, :] for b in range(BB)], axis=0)
            fwd_r = jnp.concatenate(
                [fwd_rows[b][r:r + 1, :] for b in range(BB)], axis=0)
            kdot_r = jnp.concatenate(
                [kdot_rows[b][r:r + 1, :] for b in range(BB)], axis=0)
            rk = read_keys[:, W * r:W * (r + 1)]
            kn = jnp.sqrt(jnp.sum(rk * rk, axis=1, keepdims=True))
            sc_ = (bcol(read_str[:, r:r + 1]) * kdot_r
                   / (bcol(kn) * mem_norm_new + EPS))
            m2 = jnp.max(sc_, axis=1, keepdims=True)
            e2 = jnp.exp(sc_ - bcol(m2))
            cr_r = e2 * bcol(1.0 / jnp.sum(e2, axis=1, keepdims=True))

            msl = modes_raw[:, 3 * r:3 * r + 3]
            mm = jnp.max(msl, axis=1, keepdims=True)
            me = jnp.exp(msl - mm)
            ms = me / jnp.sum(me, axis=1, keepdims=True)
            rw_r = (bcol(ms[:, 0:1]) * bwd_r + bcol(ms[:, 1:2]) * cr_r
                    + bcol(ms[:, 2:3]) * fwd_r)
            rw_new.append(rw_r)
            rw_ref[r] = rw_r

        rv_heads = [[] for _ in range(R)]
        for b in range(BB):
            rw_nb = jnp.concatenate(
                [rw_new[r][b:b + 1, :] for r in range(R)], axis=0)
            rv4 = dot(rw_nb, mem_new[b])
            for r in range(R):
                rv_heads[r].append(rv4[r:r + 1, :])
        out = dot(h, W_out_ref[0:NHID, :])
        for r in range(R):
            rv_r = jnp.concatenate(rv_heads[r], axis=0)
            rv_ref[r] = rv_r
            out = out + dot(rv_r, W_out_ref[NHID + W * r:NHID + W * (r + 1), :])
        outs_ref[t] = out + b_out_ref[...]

        prec_ref[...] = (1.0 - jnp.sum(ww, axis=1, keepdims=True)) * prec + ww
        usage_ref[...] = usage
        ww_ref[...] = ww
        h_ref[...] = h
        return 0

    lax.fori_loop(0, SEQ, step, 0)


def _run_scan(emb, h0, W_ih, W_hh, b_h, W_xi, b_xi, W_out, b_out):
    f32 = jnp.float32
    nb = BATCH // BB
    full = lambda i: (0, 0)
    return pl.pallas_call(
        _scan_kernel,
        grid=(nb,),
        in_specs=[
            pl.BlockSpec((SEQ, BB, NINP), lambda i: (0, i, 0)),
            pl.BlockSpec((BB, NHID), lambda i: (i, 0)),
            pl.BlockSpec((NINP + R * W, NHID), full),
            pl.BlockSpec((NHID, NHID), full),
            pl.BlockSpec((1, NHID), full),
            pl.BlockSpec((NHID, XI), full),
            pl.BlockSpec((1, XI), full),
            pl.BlockSpec((NHID + R * W, NINP), full),
            pl.BlockSpec((1, NINP), full),
        ],
        out_specs=pl.BlockSpec((SEQ, BB, NINP), lambda i: (0, i, 0)),
        out_shape=jax.ShapeDtypeStruct((SEQ, BATCH, NINP), f32),
        scratch_shapes=(
            [pltpu.VMEM((N, W), f32) for _ in range(BB)]
            + [pltpu.VMEM((N, N), f32) for _ in range(BB)]
            + [
                pltpu.VMEM((BB, N), f32),
                pltpu.VMEM((BB, N), f32),
                pltpu.VMEM((BB, N), f32),
                pltpu.VMEM((R, BB, N), f32),
                pltpu.VMEM((R, BB, W), f32),
                pltpu.VMEM((BB, NHID), f32),
            ]
        ),
    )(emb, h0, W_ih, W_hh, b_h, W_xi, b_xi, W_out, b_out)


def _decode_kernel(x_ref, w_ref, b_ref, o_ref):
    o_ref[...] = (jnp.dot(x_ref[...], w_ref[...],
                          preferred_element_type=jnp.float32) + b_ref[...])


def _run_decode(x2d, W_dec, b_dec):
    cb = 1280
    nb = pl.cdiv(NTOKEN, cb)
    rows = x2d.shape[0]
    return pl.pallas_call(
        _decode_kernel,
        grid=(nb,),
        in_specs=[
            pl.BlockSpec((rows, NINP), lambda j: (0, 0)),
            pl.BlockSpec((NINP, cb), lambda j: (0, j)),
            pl.BlockSpec((1, cb), lambda j: (0, j)),
        ],
        out_specs=pl.BlockSpec((rows, cb), lambda j: (0, j)),
        out_shape=jax.ShapeDtypeStruct((rows, NTOKEN), jnp.float32),
    )(x2d, W_dec, b_dec)


def kernel(input, hidden, encoder_w, W_ih, W_hh, b_h, W_xi, b_xi, W_out,
           b_out, W_dec, b_dec):
    idx_tm = jnp.transpose(input).reshape(SEQ * BATCH)
    emb2d = _sc_gather(encoder_w, idx_tm)
    emb = emb2d.reshape(SEQ, BATCH, NINP)

    outs = _run_scan(
        emb, hidden[0], W_ih, W_hh, b_h.reshape(1, NHID), W_xi,
        b_xi.reshape(1, XI), W_out, b_out.reshape(1, NINP))

    x2d = jnp.transpose(outs, (1, 0, 2)).reshape(BATCH * SEQ, NINP)
    decoded = _run_decode(x2d, W_dec, b_dec.reshape(1, NTOKEN))
    return decoded.reshape(BATCH, SEQ, NTOKEN)
